# baseline (speedup 1.0000x reference)
import functools

import jax
import jax.numpy as jnp
import numpy as np
from jax import lax
from jax.experimental import pallas as pl
from jax.experimental.pallas import tpu as pltpu
from jax.experimental.pallas import tpu_sc as plsc

NUM_TIMESTEPS = 1000
LANES = 128


def _make_table() -> np.ndarray:
    s = np.float32(0.0001)
    x = np.linspace(0.0, float(NUM_TIMESTEPS), NUM_TIMESTEPS + 1, dtype=np.float32)
    acp = np.cos((x / NUM_TIMESTEPS + s) / (1 + s) * np.float32(np.pi) * 0.5,
                 dtype=np.float32) ** 2
    acp = acp / acp[0]
    betas = (1.0 - acp[1:] / acp[:-1]).astype(np.float32)
    betas = np.clip(betas, np.float32(0.02), np.float32(0.02))
    alphas = (1.0 - betas).astype(np.float32)
    acp2 = np.cumprod(alphas, dtype=np.float32)
    table = np.zeros((NUM_TIMESTEPS, LANES), dtype=np.float32)
    table[:, 0] = np.sqrt(acp2)
    table[:, 1] = np.sqrt(1.0 - acp2)
    return table


_TABLE = _make_table()


@functools.cache
def _make_sc_gather(batch: int):
    info = plsc.get_sparse_core_info()
    num_cores = info.num_cores
    num_workers = num_cores * info.num_subcores
    b_per_w = batch // num_workers
    mesh = plsc.VectorSubcoreMesh(core_axis_name="c", subcore_axis_name="s")

    @functools.partial(
        pl.kernel,
        mesh=mesh,
        out_type=jax.ShapeDtypeStruct((batch, LANES), jnp.float32),
        scratch_types=[
            pltpu.VMEM((b_per_w,), jnp.int32),
            pltpu.VMEM((b_per_w, LANES), jnp.float32),
            pltpu.SemaphoreType.DMA,
        ],
    )
    def gather(table_hbm, ts_hbm, out_hbm, idx_v, rows_v, sem):
        wid = lax.axis_index("s") * num_cores + lax.axis_index("c")
        base = wid * b_per_w
        pltpu.sync_copy(ts_hbm.at[pl.ds(base, b_per_w)], idx_v)
        pltpu.async_copy(table_hbm.at[idx_v], rows_v, sem).wait()
        pltpu.sync_copy(rows_v, out_hbm.at[pl.ds(base, b_per_w)])

    return gather


def _combine_body(coef_ref, x_ref, n_ref, o_ref):
    c = coef_ref[...]
    a = c[:, 0:1]
    b = c[:, 1:2]
    o_ref[...] = a * x_ref[...] + b * n_ref[...]


def _combine(coefs, x2, n2, block_b: int):
    batch, feat = x2.shape
    return pl.pallas_call(
        _combine_body,
        grid=(batch // block_b,),
        in_specs=[
            pl.BlockSpec((block_b, LANES), lambda i: (i, 0)),
            pl.BlockSpec((block_b, feat), lambda i: (i, 0)),
            pl.BlockSpec((block_b, feat), lambda i: (i, 0)),
        ],
        out_specs=pl.BlockSpec((block_b, feat), lambda i: (i, 0)),
        out_shape=jax.ShapeDtypeStruct((batch, feat), jnp.float32),
    )(coefs, x2, n2)


def kernel(original_samples, noise, timesteps):
    batch = original_samples.shape[0]
    feat = int(np.prod(original_samples.shape[1:]))
    coefs = jnp.asarray(_TABLE)[timesteps]
    x2 = original_samples.reshape(batch, feat)
    n2 = noise.reshape(batch, feat)
    out = _combine(coefs, x2, n2, block_b=8)
    return out.reshape(original_samples.shape)


# speedup vs baseline: 1.1757x; 1.1757x over previous
import functools

import jax
import jax.numpy as jnp
import numpy as np
from jax import lax
from jax.experimental import pallas as pl
from jax.experimental.pallas import tpu as pltpu
from jax.experimental.pallas import tpu_sc as plsc

NUM_TIMESTEPS = 1000
LANES = 128


def _make_table() -> np.ndarray:
    s = np.float32(0.0001)
    x = np.linspace(0.0, float(NUM_TIMESTEPS), NUM_TIMESTEPS + 1, dtype=np.float32)
    acp = np.cos((x / NUM_TIMESTEPS + s) / (1 + s) * np.float32(np.pi) * 0.5,
                 dtype=np.float32) ** 2
    acp = acp / acp[0]
    betas = (1.0 - acp[1:] / acp[:-1]).astype(np.float32)
    betas = np.clip(betas, np.float32(0.02), np.float32(0.02))
    alphas = (1.0 - betas).astype(np.float32)
    acp2 = np.cumprod(alphas, dtype=np.float32)
    table = np.zeros((NUM_TIMESTEPS, LANES), dtype=np.float32)
    table[:, 0] = np.sqrt(acp2)
    table[:, 1] = np.sqrt(1.0 - acp2)
    return table


_TABLE = _make_table()


@functools.cache
def _make_sc_gather(batch: int):
    info = plsc.get_sparse_core_info()
    num_cores = info.num_cores
    num_workers = num_cores * info.num_subcores
    b_per_w = batch // num_workers
    mesh = plsc.VectorSubcoreMesh(core_axis_name="c", subcore_axis_name="s")

    @functools.partial(
        pl.kernel,
        mesh=mesh,
        out_type=jax.ShapeDtypeStruct((batch, LANES), jnp.float32),
        scratch_types=[
            pltpu.VMEM((b_per_w,), jnp.int32),
            pltpu.VMEM((b_per_w, LANES), jnp.float32),
            pltpu.SemaphoreType.DMA,
        ],
    )
    def gather(table_hbm, ts_hbm, out_hbm, idx_v, rows_v, sem):
        wid = lax.axis_index("s") * num_cores + lax.axis_index("c")
        base = wid * b_per_w
        pltpu.sync_copy(ts_hbm.at[pl.ds(base, b_per_w)], idx_v)
        pltpu.async_copy(table_hbm.at[idx_v], rows_v, sem).wait()
        pltpu.sync_copy(rows_v, out_hbm.at[pl.ds(base, b_per_w)])

    return gather


def _combine_body(coef_ref, x_ref, n_ref, o_ref):
    c = coef_ref[...]
    a = c[:, 0:1]
    b = c[:, 1:2]
    o_ref[...] = a * x_ref[...] + b * n_ref[...]


def _combine(coefs, x2, n2, block_b: int):
    batch, feat = x2.shape
    return pl.pallas_call(
        _combine_body,
        grid=(batch // block_b,),
        in_specs=[
            pl.BlockSpec((block_b, LANES), lambda i: (i, 0)),
            pl.BlockSpec((block_b, feat), lambda i: (i, 0)),
            pl.BlockSpec((block_b, feat), lambda i: (i, 0)),
        ],
        out_specs=pl.BlockSpec((block_b, feat), lambda i: (i, 0)),
        out_shape=jax.ShapeDtypeStruct((batch, feat), jnp.float32),
    )(coefs, x2, n2)


def kernel(original_samples, noise, timesteps):
    batch = original_samples.shape[0]
    feat = int(np.prod(original_samples.shape[1:]))
    coefs = jnp.asarray(_TABLE)[timesteps]
    x2 = original_samples.reshape(batch, feat)
    n2 = noise.reshape(batch, feat)
    out = _combine(coefs, x2, n2, block_b=64)
    return out.reshape(original_samples.shape)
